# baseline (speedup 1.0000x reference)
import functools

import jax
import jax.numpy as jnp
from jax import lax
from jax.experimental import pallas as pl
from jax.experimental.pallas import tpu as pltpu
from jax.experimental.pallas import tpu_sc as plsc

_INFO = plsc.get_sparse_core_info()
_NC = _INFO.num_cores
_NS = _INFO.num_subcores
_NW = _NC * _NS
_G = 16


def _make_gather(n_rows, d, batch):
    n_chunks = _NW // 2
    b_per_t = batch // n_chunks
    n_groups = b_per_t // _G
    dh = d // 2
    mesh = plsc.VectorSubcoreMesh(core_axis_name="c", subcore_axis_name="s")

    @functools.partial(
        pl.kernel,
        mesh=mesh,
        out_type=jax.ShapeDtypeStruct((d, batch), jnp.float32),
        scratch_types=[
            pltpu.VMEM((b_per_t,), jnp.int32),
            pltpu.VMEM((2, dh, _G * 128), jnp.float32),
            pltpu.VMEM((dh, b_per_t), jnp.float32),
            pltpu.SemaphoreType.DMA,
            pltpu.SemaphoreType.DMA,
        ],
        compiler_params=pltpu.CompilerParams(
            use_tc_tiling_on_sc=True, needs_layout_passes=False
        ),
    )
    def gather_kernel(idx_hbm, tab_hbm, out_hbm, idx_v, buf, cols_v, sem0, sem1):
        wid = lax.axis_index("s") * _NC + lax.axis_index("c")
        half = wid % 2
        chunk = wid // 2
        fbase = pl.multiple_of(half * dh, dh)
        cbase = pl.multiple_of(chunk * b_per_t, 128)
        pltpu.sync_copy(idx_hbm.at[pl.ds(cbase, b_per_t)], idx_v)

        feat2 = lax.bitwise_and(lax.iota(jnp.int32, _G), dh - 1)
        sel = lax.shift_right_logical(lax.iota(jnp.int32, _G), 3)

        def fire(g, slot, sem):
            idx_vec = idx_v[pl.ds(g * _G, _G)]
            col_vec = lax.shift_right_logical(idx_vec, 7) * 128
            for k in range(_G):
                q = pl.multiple_of(col_vec[k], 128)
                pltpu.async_copy(
                    tab_hbm.at[pl.ds(fbase, dh), pl.ds(q, 128)],
                    buf.at[slot].at[:, pl.ds(k * 128, 128)],
                    sem,
                )

        def drain(sem):
            pltpu.make_async_copy(
                tab_hbm.at[pl.ds(0, dh), pl.ds(0, _G * 128)], buf.at[0], sem
            ).wait()

        def extract(g, slot):
            idx_vec = idx_v[pl.ds(g * _G, _G)]
            lane_vec = lax.bitwise_and(idx_vec, 127)
            slot_idx = jnp.full((_G,), slot, jnp.int32)
            for k in range(0, _G, 2):
                m0 = jnp.broadcast_to(lane_vec[k], (_G,))
                m1 = jnp.broadcast_to(lane_vec[k + 1], (_G,))
                pos = (k + sel) * 128 + jnp.where(sel == 0, m0, m1)
                vals = plsc.load_gather(buf, [slot_idx, feat2, pos])
                plsc.store_scatter(cols_v, [feat2, g * _G + k + sel], vals)

        fire(0, 0, sem0)

        def body(j, carry):
            fire(2 * j + 1, 1, sem1)
            drain(sem0)
            extract(2 * j, 0)
            fire(2 * j + 2, 0, sem0)
            drain(sem1)
            extract(2 * j + 1, 1)
            return carry

        lax.fori_loop(0, n_groups // 2 - 1, body, 0)
        fire(n_groups - 1, 1, sem1)
        drain(sem0)
        extract(n_groups - 2, 0)
        drain(sem1)
        extract(n_groups - 1, 1)

        pltpu.sync_copy(cols_v, out_hbm.at[pl.ds(fbase, dh), pl.ds(cbase, b_per_t)])

    return gather_kernel


def kernel(states_indices, table):
    batch = states_indices.shape[0]
    n_rows, d = table.shape
    out_t = _make_gather(n_rows, d, batch)(states_indices.astype(jnp.int32), table.T)
    return out_t.T


# speedup vs baseline: 1.1633x; 1.1633x over previous
import functools

import jax
import jax.numpy as jnp
from jax import lax
from jax.experimental import pallas as pl
from jax.experimental.pallas import tpu as pltpu
from jax.experimental.pallas import tpu_sc as plsc

_INFO = plsc.get_sparse_core_info()
_NC = _INFO.num_cores
_NS = _INFO.num_subcores
_NW = _NC * _NS
_G = 16


def _make_gather(n_rows, d, batch):
    n_chunks = _NW // 2
    b_per_t = batch // n_chunks
    n_groups = b_per_t // _G
    dh = d // 2
    mesh = plsc.VectorSubcoreMesh(core_axis_name="c", subcore_axis_name="s")

    @functools.partial(
        pl.kernel,
        mesh=mesh,
        out_type=jax.ShapeDtypeStruct((d, batch), jnp.float32),
        scratch_types=[
            pltpu.VMEM((b_per_t,), jnp.int32),
            pltpu.VMEM((4, dh, _G * 128), jnp.float32),
            pltpu.VMEM((dh, b_per_t), jnp.float32),
            pltpu.SemaphoreType.DMA,
            pltpu.SemaphoreType.DMA,
            pltpu.SemaphoreType.DMA,
            pltpu.SemaphoreType.DMA,
        ],
        compiler_params=pltpu.CompilerParams(
            use_tc_tiling_on_sc=True, needs_layout_passes=False
        ),
    )
    def gather_kernel(
        idx_hbm, tab_hbm, out_hbm, idx_v, buf, cols_v, sem0, sem1, sem2, sem3
    ):
        sems = (sem0, sem1, sem2, sem3)
        wid = lax.axis_index("s") * _NC + lax.axis_index("c")
        half = wid % 2
        chunk = wid // 2
        fbase = pl.multiple_of(half * dh, dh)
        cbase = pl.multiple_of(chunk * b_per_t, 128)
        pltpu.sync_copy(idx_hbm.at[pl.ds(cbase, b_per_t)], idx_v)

        feat2 = lax.bitwise_and(lax.iota(jnp.int32, _G), dh - 1)
        sel = lax.shift_right_logical(lax.iota(jnp.int32, _G), 3)

        def fire(g, slot, sem):
            idx_vec = idx_v[pl.ds(g * _G, _G)]
            col_vec = lax.shift_right_logical(idx_vec, 7) * 128
            for k in range(_G):
                q = pl.multiple_of(col_vec[k], 128)
                pltpu.async_copy(
                    tab_hbm.at[pl.ds(fbase, dh), pl.ds(q, 128)],
                    buf.at[slot].at[:, pl.ds(k * 128, 128)],
                    sem,
                )

        def drain(sem):
            pltpu.make_async_copy(
                tab_hbm.at[pl.ds(0, dh), pl.ds(0, _G * 128)], buf.at[0], sem
            ).wait()

        def extract(g, slot):
            idx_vec = idx_v[pl.ds(g * _G, _G)]
            lane_vec = lax.bitwise_and(idx_vec, 127)
            slot_idx = jnp.full((_G,), slot, jnp.int32)
            for k in range(0, _G, 2):
                m0 = jnp.broadcast_to(lane_vec[k], (_G,))
                m1 = jnp.broadcast_to(lane_vec[k + 1], (_G,))
                pos = (k + sel) * 128 + jnp.where(sel == 0, m0, m1)
                vals = plsc.load_gather(buf, [slot_idx, feat2, pos])
                plsc.store_scatter(cols_v, [feat2, g * _G + k + sel], vals)

        fire(0, 0, sem0)
        fire(1, 1, sem1)
        fire(2, 2, sem2)

        def body(j, carry):
            for t in range(4):
                s = (3 + t) % 4
                fire(4 * j + 3 + t, s, sems[s])
                drain(sems[t])
                extract(4 * j + t, t)
            return carry

        n_body = (n_groups - 4) // 4
        lax.fori_loop(0, n_body, body, 0)
        fire(n_groups - 1, (n_groups - 1) % 4, sems[(n_groups - 1) % 4])
        for g in range(n_groups - 4, n_groups):
            drain(sems[g % 4])
            extract(g, g % 4)

        pltpu.sync_copy(cols_v, out_hbm.at[pl.ds(fbase, dh), pl.ds(cbase, b_per_t)])

    return gather_kernel


def kernel(states_indices, table):
    batch = states_indices.shape[0]
    n_rows, d = table.shape
    out_t = _make_gather(n_rows, d, batch)(states_indices.astype(jnp.int32), table.T)
    return out_t.T


# speedup vs baseline: 1.1849x; 1.0186x over previous
import functools

import jax
import jax.numpy as jnp
from jax import lax
from jax.experimental import pallas as pl
from jax.experimental.pallas import tpu as pltpu
from jax.experimental.pallas import tpu_sc as plsc

_INFO = plsc.get_sparse_core_info()
_NC = _INFO.num_cores
_NS = _INFO.num_subcores
_NW = _NC * _NS
_G = 32
_H = 16


def _make_gather(n_rows, d, batch):
    n_chunks = _NW // 2
    b_per_t = batch // n_chunks
    n_groups = b_per_t // _G
    dh = d // 2
    mesh = plsc.VectorSubcoreMesh(core_axis_name="c", subcore_axis_name="s")

    @functools.partial(
        pl.kernel,
        mesh=mesh,
        out_type=jax.ShapeDtypeStruct((d, batch), jnp.float32),
        scratch_types=[
            pltpu.VMEM((b_per_t,), jnp.int32),
            pltpu.VMEM((3, dh, _G * 128), jnp.float32),
            pltpu.VMEM((dh, b_per_t), jnp.float32),
            pltpu.SemaphoreType.DMA,
            pltpu.SemaphoreType.DMA,
            pltpu.SemaphoreType.DMA,
        ],
        compiler_params=pltpu.CompilerParams(
            use_tc_tiling_on_sc=True, needs_layout_passes=False
        ),
    )
    def gather_kernel(
        idx_hbm, tab_hbm, out_hbm, idx_v, buf, cols_v, sem0, sem1, sem2
    ):
        sems = (sem0, sem1, sem2)
        wid = lax.axis_index("s") * _NC + lax.axis_index("c")
        half = wid % 2
        chunk = wid // 2
        fbase = pl.multiple_of(half * dh, dh)
        cbase = pl.multiple_of(chunk * b_per_t, 128)
        pltpu.sync_copy(idx_hbm.at[pl.ds(cbase, b_per_t)], idx_v)

        feat2 = lax.bitwise_and(lax.iota(jnp.int32, _H), dh - 1)
        sel = lax.shift_right_logical(lax.iota(jnp.int32, _H), 3)

        def fire(g, slot, sem):
            for off in range(0, _G, _H):
                idx_vec = idx_v[pl.ds(g * _G + off, _H)]
                col_vec = lax.shift_right_logical(idx_vec, 7) * 128
                for k in range(_H):
                    q = pl.multiple_of(col_vec[k], 128)
                    pltpu.async_copy(
                        tab_hbm.at[pl.ds(fbase, dh), pl.ds(q, 128)],
                        buf.at[slot].at[:, pl.ds((off + k) * 128, 128)],
                        sem,
                    )

        def drain(sem):
            pltpu.make_async_copy(
                tab_hbm.at[pl.ds(0, dh), pl.ds(0, _G * 128)], buf.at[0], sem
            ).wait()

        def extract(g, slot):
            slot_idx = jnp.full((_H,), slot, jnp.int32)
            for off in range(0, _G, _H):
                idx_vec = idx_v[pl.ds(g * _G + off, _H)]
                lane_vec = lax.bitwise_and(idx_vec, 127)
                for k in range(0, _H, 2):
                    m0 = jnp.broadcast_to(lane_vec[k], (_H,))
                    m1 = jnp.broadcast_to(lane_vec[k + 1], (_H,))
                    pos = (off + k + sel) * 128 + jnp.where(sel == 0, m0, m1)
                    vals = plsc.load_gather(buf, [slot_idx, feat2, pos])
                    plsc.store_scatter(
                        cols_v, [feat2, g * _G + off + k + sel], vals
                    )

        fire(0, 0, sem0)
        fire(1, 1, sem1)

        def body(j, carry):
            for t in range(3):
                s = (2 + t) % 3
                fire(3 * j + 2 + t, s, sems[s])
                drain(sems[t])
                extract(3 * j + t, t)
            return carry

        n_body = (n_groups - 2) // 3
        lax.fori_loop(0, n_body, body, 0)
        for g in range(n_groups - 2, n_groups):
            drain(sems[g % 3])
            extract(g, g % 3)

        pltpu.sync_copy(cols_v, out_hbm.at[pl.ds(fbase, dh), pl.ds(cbase, b_per_t)])

    return gather_kernel


def kernel(states_indices, table):
    batch = states_indices.shape[0]
    n_rows, d = table.shape
    out_t = _make_gather(n_rows, d, batch)(states_indices.astype(jnp.int32), table.T)
    return out_t.T
